# baseline (speedup 1.0000x reference)
import functools

import jax
import jax.numpy as jnp
from jax import lax
from jax.experimental import pallas as pl
from jax.experimental.pallas import tpu as pltpu
from jax.experimental.pallas import tpu_sc as plsc

_N = 10000
_E = 320000
_D = 128
_NC = 2
_NS = 16
_NW = _NC * _NS
_CHUNK = 128
_NCH = 80
_EPAD = _NW * _NCH * _CHUNK
_NACC = 10240
_RPT = _NACC // _NS

_VMESH = plsc.VectorSubcoreMesh(core_axis_name="c", subcore_axis_name="s")


@functools.partial(
    pl.kernel, mesh=_VMESH,
    out_type=jax.ShapeDtypeStruct((_NC, _NACC, _D), jnp.float32),
    scratch_types=[
        pltpu.VMEM((_NCH, _CHUNK), jnp.int32),
        pltpu.VMEM((_CHUNK,), jnp.int32),
        pltpu.VMEM((_CHUNK,), jnp.int32),
        pltpu.VMEM((_CHUNK, _D), jnp.float32),
        pltpu.VMEM((_CHUNK, _D), jnp.float32),
        pltpu.VMEM_SHARED((_NACC, _D), jnp.float32),
        pltpu.SemaphoreType.DMA,
        pltpu.SemaphoreType.DMA,
        pltpu.SemaphoreType.DMA,
        pltpu.SemaphoreType.DMA,
    ])
def _sc_segsum(g_h, src_h, dst_h, zeros_h, p_h, src_v, dst_a, dst_b,
               rows_a, rows_b, acc, sem_a, sem_b, sem_da, sem_db):
    c = lax.axis_index("c")
    s = lax.axis_index("s")
    wid = c * _NS + s
    r0 = s * _RPT
    pltpu.sync_copy(zeros_h.at[pl.ds(r0, _RPT)], acc.at[pl.ds(r0, _RPT)])
    pltpu.sync_copy(src_h.at[wid], src_v)
    plsc.subcore_barrier()

    pltpu.async_copy(g_h.at[src_v.at[0]], rows_a, sem_a)
    pltpu.async_copy(g_h.at[src_v.at[1]], rows_b, sem_b)
    pltpu.async_copy(dst_h.at[wid, 0], dst_a, sem_da)
    pltpu.async_copy(dst_h.at[wid, 1], dst_b, sem_db)

    @pl.loop(0, _NCH // 2 - 1)
    def _(i):
        j = 2 * i
        pltpu.make_async_copy(g_h.at[src_v.at[j]], rows_a, sem_a).wait()
        pltpu.make_async_copy(dst_h.at[wid, j], dst_a, sem_da).wait()
        pltpu.sync_copy(rows_a, acc.at[dst_a], add=True)
        pltpu.async_copy(g_h.at[src_v.at[j + 2]], rows_a, sem_a)
        pltpu.async_copy(dst_h.at[wid, j + 2], dst_a, sem_da)
        pltpu.make_async_copy(g_h.at[src_v.at[j + 1]], rows_b, sem_b).wait()
        pltpu.make_async_copy(dst_h.at[wid, j + 1], dst_b, sem_db).wait()
        pltpu.sync_copy(rows_b, acc.at[dst_b], add=True)
        pltpu.async_copy(g_h.at[src_v.at[j + 3]], rows_b, sem_b)
        pltpu.async_copy(dst_h.at[wid, j + 3], dst_b, sem_db)

    pltpu.make_async_copy(g_h.at[src_v.at[_NCH - 2]], rows_a, sem_a).wait()
    pltpu.make_async_copy(dst_h.at[wid, _NCH - 2], dst_a, sem_da).wait()
    pltpu.sync_copy(rows_a, acc.at[dst_a], add=True)
    pltpu.make_async_copy(g_h.at[src_v.at[_NCH - 1]], rows_b, sem_b).wait()
    pltpu.make_async_copy(dst_h.at[wid, _NCH - 1], dst_b, sem_db).wait()
    pltpu.sync_copy(rows_b, acc.at[dst_b], add=True)

    plsc.subcore_barrier()
    pltpu.sync_copy(acc.at[pl.ds(r0, _RPT)], p_h.at[c, pl.ds(r0, _RPT)])


@functools.partial(
    pl.kernel, mesh=_VMESH,
    out_type=jax.ShapeDtypeStruct((_NC, _NACC, _D), jnp.float32),
    scratch_types=[
        pltpu.VMEM((_NCH, _CHUNK), jnp.int32),
        pltpu.VMEM((_CHUNK, _D), jnp.float32),
        pltpu.VMEM_SHARED((_NACC, _D), jnp.float32),
        pltpu.SemaphoreType.DMA,
    ])
def _sc_degree(dst_h, zeros_h, ones_h, dp_h, dst_v, ones_v, acc, sem):
    c = lax.axis_index("c")
    s = lax.axis_index("s")
    wid = c * _NS + s
    r0 = s * _RPT
    pltpu.sync_copy(zeros_h.at[pl.ds(r0, _RPT)], acc.at[pl.ds(r0, _RPT)])
    pltpu.sync_copy(ones_h, ones_v)
    pltpu.sync_copy(dst_h.at[wid], dst_v)
    plsc.subcore_barrier()

    @pl.loop(0, _NCH)
    def _(j):
        pltpu.sync_copy(ones_v, acc.at[dst_v.at[j]], add=True)

    plsc.subcore_barrier()
    pltpu.sync_copy(acc.at[pl.ds(r0, _RPT)], dp_h.at[c, pl.ds(r0, _RPT)])


_BLK = 1000
_GRID = _N // _BLK


def _tc_matmul(x, w):
    dout = w.shape[1]

    def body(x_ref, w_ref, o_ref):
        o_ref[...] = jnp.dot(x_ref[...], w_ref[...],
                             preferred_element_type=jnp.float32)

    return pl.pallas_call(
        body,
        grid=(_GRID,),
        in_specs=[
            pl.BlockSpec((_BLK, x.shape[1]), lambda i: (i, 0)),
            pl.BlockSpec((x.shape[1], dout), lambda i: (0, 0)),
        ],
        out_specs=pl.BlockSpec((_BLK, dout), lambda i: (i, 0)),
        out_shape=jax.ShapeDtypeStruct((_N, dout), jnp.float32),
    )(x, w)


def _tc_combine(h, p, degp, wself, b, relu, wneigh_next=None, wneigh_s=None):
    dout = wself.shape[1]
    pw = p.shape[2]
    dw = degp.shape[2]
    b2 = b.reshape(1, dout)

    def body(h_ref, p0_ref, p1_ref, d0_ref, d1_ref, ws_ref, b_ref, *rest):
        rest = list(rest)
        wns_ref = rest.pop(0) if wneigh_s is not None else None
        wn_ref = rest.pop(0) if wneigh_next is not None else None
        o_ref = rest.pop(0)
        g_ref = rest.pop(0) if wneigh_next is not None else None
        deg = d0_ref[0, :, 0:1] + d1_ref[0, :, 0:1]
        rdeg = 1.0 / jnp.maximum(deg, 1.0)
        mean = (p0_ref[0] + p1_ref[0]) * rdeg
        if wns_ref is not None:
            mean = jnp.dot(mean, wns_ref[...],
                           preferred_element_type=jnp.float32)
        z = jnp.dot(h_ref[...], ws_ref[...],
                    preferred_element_type=jnp.float32) + mean + b_ref[...]
        if relu:
            z = jnp.maximum(z, 0.0)
        o_ref[...] = z
        if g_ref is not None:
            g_ref[...] = jnp.dot(z, wn_ref[...],
                                 preferred_element_type=jnp.float32)

    in_specs = [
        pl.BlockSpec((_BLK, h.shape[1]), lambda i: (i, 0)),
        pl.BlockSpec((1, _BLK, pw), lambda i: (0, i, 0)),
        pl.BlockSpec((1, _BLK, pw), lambda i: (1, i, 0)),
        pl.BlockSpec((1, _BLK, dw), lambda i: (0, i, 0)),
        pl.BlockSpec((1, _BLK, dw), lambda i: (1, i, 0)),
        pl.BlockSpec((h.shape[1], dout), lambda i: (0, 0)),
        pl.BlockSpec((1, dout), lambda i: (0, 0)),
    ]
    args = [h, p, p, degp, degp, wself, b2]
    if wneigh_s is not None:
        in_specs.append(pl.BlockSpec(wneigh_s.shape, lambda i: (0, 0)))
        args.append(wneigh_s)
    out_shapes = [jax.ShapeDtypeStruct((_N, dout), jnp.float32)]
    out_specs = [pl.BlockSpec((_BLK, dout), lambda i: (i, 0))]
    if wneigh_next is not None:
        dnext = wneigh_next.shape[1]
        in_specs.append(pl.BlockSpec((dout, dnext), lambda i: (0, 0)))
        args.append(wneigh_next)
        out_shapes.append(jax.ShapeDtypeStruct((_N, dnext), jnp.float32))
        out_specs.append(pl.BlockSpec((_BLK, dnext), lambda i: (i, 0)))

    return pl.pallas_call(
        body,
        grid=(_GRID,),
        in_specs=in_specs,
        out_specs=out_specs,
        out_shape=out_shapes,
    )(*args)


def kernel(x, edge_index, Wself0, Wneigh0, b0, Wself1, Wneigh1, b1,
           Wself2, Wneigh2, b2):
    src = edge_index[0]
    dst = edge_index[1]
    npad = _EPAD - _E
    src_c = jnp.concatenate([src, jnp.zeros((npad,), jnp.int32)])
    src_c = src_c.reshape(_NW, _NCH, _CHUNK)
    dst_c = jnp.concatenate([dst, jnp.full((npad,), _N, jnp.int32)])
    dst_c = dst_c.reshape(_NW, _NCH, _CHUNK)
    zeros = jnp.zeros((_NACC, _D), jnp.float32)
    ones_rows = jnp.ones((_CHUNK, _D), jnp.float32)

    degp = _sc_degree(dst_c, zeros, ones_rows)

    g0 = _tc_matmul(x, Wneigh0)
    p0 = _sc_segsum(g0, src_c, dst_c, zeros)
    h1, g1 = _tc_combine(x, p0, degp, Wself0, b0, True, wneigh_next=Wneigh1)

    p1 = _sc_segsum(g1, src_c, dst_c, zeros)
    (h2,) = _tc_combine(h1, p1, degp, Wself1, b1, True)

    p2 = _sc_segsum(h2, src_c, dst_c, zeros)
    out = _tc_combine(h2, p2, degp, Wself2, b2, False, wneigh_s=Wneigh2)
    return out[0]


# speedup vs baseline: 1.0867x; 1.0867x over previous
import functools

import jax
import jax.numpy as jnp
from jax import lax
from jax.experimental import pallas as pl
from jax.experimental.pallas import tpu as pltpu
from jax.experimental.pallas import tpu_sc as plsc

_N = 10000
_E = 320000
_D = 128
_NC = 2
_NS = 16
_NW = _NC * _NS
_CHUNK = 128
_NCH = 80
_KF = 128
_KS = 32
_FAST_CORE = 0
_TOTCH = _NS * (_KF + _KS)
_TOTCH_P = _TOTCH + (_KF - _KS)
_EPAD = _TOTCH * _CHUNK
_NACC = 10112
_RPT = _NACC // _NS

_VMESH = plsc.VectorSubcoreMesh(core_axis_name="c", subcore_axis_name="s")


@functools.partial(
    pl.kernel, mesh=_VMESH,
    out_type=jax.ShapeDtypeStruct((_NC, _NACC, _D), jnp.float32),
    scratch_types=[
        pltpu.VMEM((_KF, _CHUNK), jnp.int32),
        pltpu.VMEM((_CHUNK,), jnp.int32),
        pltpu.VMEM((_CHUNK,), jnp.int32),
        pltpu.VMEM((_CHUNK, _D), jnp.float32),
        pltpu.VMEM((_CHUNK, _D), jnp.float32),
        pltpu.VMEM_SHARED((_NACC, _D), jnp.float32),
        pltpu.SemaphoreType.DMA,
        pltpu.SemaphoreType.DMA,
        pltpu.SemaphoreType.DMA,
        pltpu.SemaphoreType.DMA,
    ])
def _sc_segsum(g_h, src_h, dst_h, zeros_h, p_h, src_v, dst_a, dst_b,
               rows_a, rows_b, acc, sem_a, sem_b, sem_da, sem_db):
    c = lax.axis_index("c")
    s = lax.axis_index("s")
    r0 = s * _RPT
    is_fast = c == _FAST_CORE
    k = jnp.where(is_fast, _KF, _KS)
    off = jnp.where(is_fast, s * _KF, _NS * _KF + s * _KS)
    pltpu.sync_copy(zeros_h.at[pl.ds(r0, _RPT)], acc.at[pl.ds(r0, _RPT)])
    pltpu.sync_copy(src_h.at[pl.ds(off, _KF)], src_v)
    plsc.subcore_barrier()

    pltpu.async_copy(g_h.at[src_v.at[0]], rows_a, sem_a)
    pltpu.async_copy(g_h.at[src_v.at[1]], rows_b, sem_b)
    pltpu.async_copy(dst_h.at[off], dst_a, sem_da)
    pltpu.async_copy(dst_h.at[off + 1], dst_b, sem_db)

    @pl.loop(0, k // 2 - 1)
    def _(i):
        j = 2 * i
        pltpu.make_async_copy(g_h.at[src_v.at[j]], rows_a, sem_a).wait()
        pltpu.make_async_copy(dst_h.at[off + j], dst_a, sem_da).wait()
        pltpu.sync_copy(rows_a, acc.at[dst_a], add=True)
        pltpu.async_copy(g_h.at[src_v.at[j + 2]], rows_a, sem_a)
        pltpu.async_copy(dst_h.at[off + j + 2], dst_a, sem_da)
        pltpu.make_async_copy(g_h.at[src_v.at[j + 1]], rows_b, sem_b).wait()
        pltpu.make_async_copy(dst_h.at[off + j + 1], dst_b, sem_db).wait()
        pltpu.sync_copy(rows_b, acc.at[dst_b], add=True)
        pltpu.async_copy(g_h.at[src_v.at[j + 3]], rows_b, sem_b)
        pltpu.async_copy(dst_h.at[off + j + 3], dst_b, sem_db)

    pltpu.make_async_copy(g_h.at[src_v.at[k - 2]], rows_a, sem_a).wait()
    pltpu.make_async_copy(dst_h.at[off + k - 2], dst_a, sem_da).wait()
    pltpu.sync_copy(rows_a, acc.at[dst_a], add=True)
    pltpu.make_async_copy(g_h.at[src_v.at[k - 1]], rows_b, sem_b).wait()
    pltpu.make_async_copy(dst_h.at[off + k - 1], dst_b, sem_db).wait()
    pltpu.sync_copy(rows_b, acc.at[dst_b], add=True)

    plsc.subcore_barrier()
    pltpu.sync_copy(acc.at[pl.ds(r0, _RPT)], p_h.at[c, pl.ds(r0, _RPT)])


@functools.partial(
    pl.kernel, mesh=_VMESH,
    out_type=jax.ShapeDtypeStruct((_NC, _NACC, _D), jnp.float32),
    scratch_types=[
        pltpu.VMEM((_NCH, _CHUNK), jnp.int32),
        pltpu.VMEM((_CHUNK, _D), jnp.float32),
        pltpu.VMEM_SHARED((_NACC, _D), jnp.float32),
        pltpu.SemaphoreType.DMA,
    ])
def _sc_degree(dst_h, zeros_h, ones_h, dp_h, dst_v, ones_v, acc, sem):
    c = lax.axis_index("c")
    s = lax.axis_index("s")
    wid = c * _NS + s
    r0 = s * _RPT
    pltpu.sync_copy(zeros_h.at[pl.ds(r0, _RPT)], acc.at[pl.ds(r0, _RPT)])
    pltpu.sync_copy(ones_h, ones_v)
    pltpu.sync_copy(dst_h.at[wid], dst_v)
    plsc.subcore_barrier()

    @pl.loop(0, _NCH)
    def _(j):
        pltpu.sync_copy(ones_v, acc.at[dst_v.at[j]], add=True)

    plsc.subcore_barrier()
    pltpu.sync_copy(acc.at[pl.ds(r0, _RPT)], dp_h.at[c, pl.ds(r0, _RPT)])


_BLK = 1000
_GRID = _N // _BLK


def _tc_matmul(x, w):
    dout = w.shape[1]

    def body(x_ref, w_ref, o_ref):
        o_ref[...] = jnp.dot(x_ref[...], w_ref[...],
                             preferred_element_type=jnp.float32)

    return pl.pallas_call(
        body,
        grid=(_GRID,),
        in_specs=[
            pl.BlockSpec((_BLK, x.shape[1]), lambda i: (i, 0)),
            pl.BlockSpec((x.shape[1], dout), lambda i: (0, 0)),
        ],
        out_specs=pl.BlockSpec((_BLK, dout), lambda i: (i, 0)),
        out_shape=jax.ShapeDtypeStruct((_N, dout), jnp.float32),
    )(x, w)


def _tc_combine(h, p, degp, wself, b, relu, wneigh_next=None, wneigh_s=None):
    dout = wself.shape[1]
    pw = p.shape[2]
    dw = degp.shape[2]
    b2 = b.reshape(1, dout)

    def body(h_ref, p0_ref, p1_ref, d0_ref, d1_ref, ws_ref, b_ref, *rest):
        rest = list(rest)
        wns_ref = rest.pop(0) if wneigh_s is not None else None
        wn_ref = rest.pop(0) if wneigh_next is not None else None
        o_ref = rest.pop(0)
        g_ref = rest.pop(0) if wneigh_next is not None else None
        deg = d0_ref[0, :, 0:1] + d1_ref[0, :, 0:1]
        rdeg = 1.0 / jnp.maximum(deg, 1.0)
        mean = (p0_ref[0] + p1_ref[0]) * rdeg
        if wns_ref is not None:
            mean = jnp.dot(mean, wns_ref[...],
                           preferred_element_type=jnp.float32)
        z = jnp.dot(h_ref[...], ws_ref[...],
                    preferred_element_type=jnp.float32) + mean + b_ref[...]
        if relu:
            z = jnp.maximum(z, 0.0)
        o_ref[...] = z
        if g_ref is not None:
            g_ref[...] = jnp.dot(z, wn_ref[...],
                                 preferred_element_type=jnp.float32)

    in_specs = [
        pl.BlockSpec((_BLK, h.shape[1]), lambda i: (i, 0)),
        pl.BlockSpec((1, _BLK, pw), lambda i: (0, i, 0)),
        pl.BlockSpec((1, _BLK, pw), lambda i: (1, i, 0)),
        pl.BlockSpec((1, _BLK, dw), lambda i: (0, i, 0)),
        pl.BlockSpec((1, _BLK, dw), lambda i: (1, i, 0)),
        pl.BlockSpec((h.shape[1], dout), lambda i: (0, 0)),
        pl.BlockSpec((1, dout), lambda i: (0, 0)),
    ]
    args = [h, p, p, degp, degp, wself, b2]
    if wneigh_s is not None:
        in_specs.append(pl.BlockSpec(wneigh_s.shape, lambda i: (0, 0)))
        args.append(wneigh_s)
    out_shapes = [jax.ShapeDtypeStruct((_N, dout), jnp.float32)]
    out_specs = [pl.BlockSpec((_BLK, dout), lambda i: (i, 0))]
    if wneigh_next is not None:
        dnext = wneigh_next.shape[1]
        in_specs.append(pl.BlockSpec((dout, dnext), lambda i: (0, 0)))
        args.append(wneigh_next)
        out_shapes.append(jax.ShapeDtypeStruct((_N, dnext), jnp.float32))
        out_specs.append(pl.BlockSpec((_BLK, dnext), lambda i: (i, 0)))

    return pl.pallas_call(
        body,
        grid=(_GRID,),
        in_specs=in_specs,
        out_specs=out_specs,
        out_shape=out_shapes,
    )(*args)


def kernel(x, edge_index, Wself0, Wneigh0, b0, Wself1, Wneigh1, b1,
           Wself2, Wneigh2, b2):
    src = edge_index[0]
    dst = edge_index[1]
    npad = _EPAD - _E
    src_d = jnp.concatenate([src, jnp.zeros((npad,), jnp.int32)])
    dst_d = jnp.concatenate([dst, jnp.full((npad,), _N, jnp.int32)])
    tpad = (_TOTCH_P - _TOTCH) * _CHUNK
    src_c = jnp.concatenate(
        [src_d, jnp.zeros((tpad,), jnp.int32)]).reshape(_TOTCH_P, _CHUNK)
    dst_c = jnp.concatenate(
        [dst_d, jnp.full((tpad,), _N, jnp.int32)]).reshape(_TOTCH_P, _CHUNK)
    dst_w = dst_d.reshape(_NW, _NCH, _CHUNK)
    zeros = jnp.zeros((_NACC, _D), jnp.float32)
    ones_rows = jnp.ones((_CHUNK, _D), jnp.float32)

    degp = _sc_degree(dst_w, zeros, ones_rows)

    g0 = _tc_matmul(x, Wneigh0)
    p0 = _sc_segsum(g0, src_c, dst_c, zeros)
    h1, g1 = _tc_combine(x, p0, degp, Wself0, b0, True, wneigh_next=Wneigh1)

    p1 = _sc_segsum(g1, src_c, dst_c, zeros)
    (h2,) = _tc_combine(h1, p1, degp, Wself1, b1, True)

    p2 = _sc_segsum(h2, src_c, dst_c, zeros)
    out = _tc_combine(h2, p2, degp, Wself2, b2, False, wneigh_s=Wneigh2)
    return out[0]


# speedup vs baseline: 1.1333x; 1.0429x over previous
import functools

import jax
import jax.numpy as jnp
from jax import lax
from jax.experimental import pallas as pl
from jax.experimental.pallas import tpu as pltpu
from jax.experimental.pallas import tpu_sc as plsc

_N = 10000
_E = 320000
_D = 128
_NC = 2
_NS = 16
_NW = _NC * _NS
_CHUNK = 128
_EPW = 10112
_NCHUNK = _EPW // _CHUNK
_EPAD = _EPW * _NW
_NACC = 10240
_RPT = _NACC // _NS

_VMESH = plsc.VectorSubcoreMesh(core_axis_name="c", subcore_axis_name="s")


@functools.partial(
    pl.kernel, mesh=_VMESH,
    out_type=jax.ShapeDtypeStruct((_NC, _NACC, _D), jnp.float32),
    scratch_types=[
        pltpu.VMEM((_CHUNK,), jnp.int32),
        pltpu.VMEM((_CHUNK,), jnp.int32),
        pltpu.VMEM((_CHUNK, _D), jnp.float32),
        pltpu.VMEM_SHARED((_NACC, _D), jnp.float32),
        pltpu.SemaphoreType.DMA,
    ])
def _sc_segsum(g_h, src_h, dst_h, zeros_h, p_h, idx_s, idx_d, rows_v, acc, sem):
    c = lax.axis_index("c")
    s = lax.axis_index("s")
    wid = c * _NS + s
    r0 = s * _RPT
    pltpu.sync_copy(zeros_h.at[pl.ds(r0, _RPT)], acc.at[pl.ds(r0, _RPT)])
    plsc.subcore_barrier()

    base0 = wid * _EPW

    @pl.loop(0, _NCHUNK)
    def _(j):
        base = base0 + j * _CHUNK
        pltpu.sync_copy(src_h.at[pl.ds(base, _CHUNK)], idx_s)
        pltpu.sync_copy(dst_h.at[pl.ds(base, _CHUNK)], idx_d)
        pltpu.async_copy(g_h.at[idx_s], rows_v, sem).wait()
        pltpu.sync_copy(rows_v, acc.at[idx_d], add=True)

    plsc.subcore_barrier()
    pltpu.sync_copy(acc.at[pl.ds(r0, _RPT)], p_h.at[c, pl.ds(r0, _RPT)])


@functools.partial(
    pl.kernel, mesh=_VMESH,
    out_type=jax.ShapeDtypeStruct((_NC, _NACC, _D), jnp.float32),
    scratch_types=[
        pltpu.VMEM((_CHUNK,), jnp.int32),
        pltpu.VMEM((_CHUNK, _D), jnp.float32),
        pltpu.VMEM_SHARED((_NACC, _D), jnp.float32),
        pltpu.SemaphoreType.DMA,
    ])
def _sc_degree(dst_h, zeros_h, ones_h, dp_h, idx_d, ones_v, acc, sem):
    c = lax.axis_index("c")
    s = lax.axis_index("s")
    wid = c * _NS + s
    r0 = s * _RPT
    pltpu.sync_copy(zeros_h.at[pl.ds(r0, _RPT)], acc.at[pl.ds(r0, _RPT)])
    pltpu.sync_copy(ones_h, ones_v)
    plsc.subcore_barrier()

    base0 = wid * _EPW

    @pl.loop(0, _NCHUNK)
    def _(j):
        base = base0 + j * _CHUNK
        pltpu.sync_copy(dst_h.at[pl.ds(base, _CHUNK)], idx_d)
        pltpu.sync_copy(ones_v, acc.at[idx_d], add=True)

    plsc.subcore_barrier()
    pltpu.sync_copy(acc.at[pl.ds(r0, _RPT)], dp_h.at[c, pl.ds(r0, _RPT)])


_BLK = 1000
_GRID = _N // _BLK


def _tc_matmul(x, w):
    dout = w.shape[1]

    def body(x_ref, w_ref, o_ref):
        o_ref[...] = jnp.dot(x_ref[...], w_ref[...],
                             preferred_element_type=jnp.float32)

    return pl.pallas_call(
        body,
        grid=(_GRID,),
        in_specs=[
            pl.BlockSpec((_BLK, x.shape[1]), lambda i: (i, 0)),
            pl.BlockSpec((x.shape[1], dout), lambda i: (0, 0)),
        ],
        out_specs=pl.BlockSpec((_BLK, dout), lambda i: (i, 0)),
        out_shape=jax.ShapeDtypeStruct((_N, dout), jnp.float32),
    )(x, w)


def _tc_combine(h, p, degp, wself, b, relu, wneigh_next=None, wneigh_s=None):
    dout = wself.shape[1]
    b2 = b.reshape(1, dout)

    def body(h_ref, p0_ref, p1_ref, d0_ref, d1_ref, ws_ref, b_ref, *rest):
        rest = list(rest)
        wns_ref = rest.pop(0) if wneigh_s is not None else None
        wn_ref = rest.pop(0) if wneigh_next is not None else None
        o_ref = rest.pop(0)
        g_ref = rest.pop(0) if wneigh_next is not None else None
        deg = d0_ref[0, :, 0:1] + d1_ref[0, :, 0:1]
        rdeg = 1.0 / jnp.maximum(deg, 1.0)
        mean = (p0_ref[0] + p1_ref[0]) * rdeg
        if wns_ref is not None:
            mean = jnp.dot(mean, wns_ref[...],
                           preferred_element_type=jnp.float32)
        z = jnp.dot(h_ref[...], ws_ref[...],
                    preferred_element_type=jnp.float32) + mean + b_ref[...]
        if relu:
            z = jnp.maximum(z, 0.0)
        o_ref[...] = z
        if g_ref is not None:
            g_ref[...] = jnp.dot(z, wn_ref[...],
                                 preferred_element_type=jnp.float32)

    in_specs = [
        pl.BlockSpec((_BLK, h.shape[1]), lambda i: (i, 0)),
        pl.BlockSpec((1, _BLK, _D), lambda i: (0, i, 0)),
        pl.BlockSpec((1, _BLK, _D), lambda i: (1, i, 0)),
        pl.BlockSpec((1, _BLK, _D), lambda i: (0, i, 0)),
        pl.BlockSpec((1, _BLK, _D), lambda i: (1, i, 0)),
        pl.BlockSpec((h.shape[1], dout), lambda i: (0, 0)),
        pl.BlockSpec((1, dout), lambda i: (0, 0)),
    ]
    args = [h, p, p, degp, degp, wself, b2]
    if wneigh_s is not None:
        in_specs.append(pl.BlockSpec(wneigh_s.shape, lambda i: (0, 0)))
        args.append(wneigh_s)
    out_shapes = [jax.ShapeDtypeStruct((_N, dout), jnp.float32)]
    out_specs = [pl.BlockSpec((_BLK, dout), lambda i: (i, 0))]
    if wneigh_next is not None:
        dnext = wneigh_next.shape[1]
        in_specs.append(pl.BlockSpec((dout, dnext), lambda i: (0, 0)))
        args.append(wneigh_next)
        out_shapes.append(jax.ShapeDtypeStruct((_N, dnext), jnp.float32))
        out_specs.append(pl.BlockSpec((_BLK, dnext), lambda i: (i, 0)))

    return pl.pallas_call(
        body,
        grid=(_GRID,),
        in_specs=in_specs,
        out_specs=out_specs,
        out_shape=out_shapes,
    )(*args)


def kernel(x, edge_index, Wself0, Wneigh0, b0, Wself1, Wneigh1, b1,
           Wself2, Wneigh2, b2):
    src = edge_index[0]
    dst = edge_index[1]
    npad = _EPAD - _E
    src_p = jnp.concatenate([src, jnp.zeros((npad,), jnp.int32)])
    dst_p = jnp.concatenate([dst, jnp.full((npad,), _N, jnp.int32)])
    zeros = jnp.zeros((_NACC, _D), jnp.float32)
    ones_rows = jnp.ones((_CHUNK, _D), jnp.float32)

    degp = _sc_degree(dst_p, zeros, ones_rows)

    g0 = _tc_matmul(x, Wneigh0)
    p0 = _sc_segsum(g0, src_p, dst_p, zeros)
    h1, g1 = _tc_combine(x, p0, degp, Wself0, b0, True, wneigh_next=Wneigh1)

    p1 = _sc_segsum(g1, src_p, dst_p, zeros)
    (h2,) = _tc_combine(h1, p1, degp, Wself1, b1, True)

    p2 = _sc_segsum(h2, src_p, dst_p, zeros)
    out = _tc_combine(h2, p2, degp, Wself2, b2, False, wneigh_s=Wneigh2)
    return out[0]
